# Initial kernel scaffold; baseline (speedup 1.0000x reference)
import jax
import jax.numpy as jnp
from jax.experimental import pallas as pl


def kernel(product, customer, color, size, group, W_product, W_customer, W_color, W_size, W_group):
    raise NotImplementedError("write your pallas kernel here")


# speedup vs baseline: 1.4815x; 1.4815x over previous
import functools

import jax
import jax.numpy as jnp
from jax import lax
from jax.experimental import pallas as pl
from jax.experimental.pallas import tpu as pltpu
from jax.experimental.pallas import tpu_sc as plsc

_NC, _NS = 2, 16
_NW = _NC * _NS
_B = 1024 * 20
_PER_W = _B // _NW
_CHUNK = 64
_NCHUNK = _PER_W // _CHUNK

_D_BIG = 768
_D_SMALL = 256
_D_OUT = 2 * _D_BIG + 3 * _D_SMALL


def _build():
  mesh = plsc.VectorSubcoreMesh(core_axis_name="c", subcore_axis_name="s")

  @functools.partial(
      pl.kernel, mesh=mesh,
      out_type=jax.ShapeDtypeStruct((_B, _D_OUT), jnp.float32),
      scratch_types=[
          pltpu.VMEM((_NCHUNK, _CHUNK), jnp.int32),
          pltpu.VMEM((_CHUNK, _D_BIG), jnp.float32),
          pltpu.VMEM((_CHUNK, _D_SMALL), jnp.float32),
          pltpu.SemaphoreType.DMA,
      ],
  )
  def emb(ip, ic, icol, isz, igr, wp, wc, wcol, wsz, wgr,
          out, idx_v, big_v, small_v, sem):
    wid = lax.axis_index("s") * _NC + lax.axis_index("c")
    base = wid * _PER_W
    tables = [
        (ip, wp, big_v, _D_BIG, 0),
        (ic, wc, big_v, _D_BIG, _D_BIG),
        (icol, wcol, small_v, _D_SMALL, 2 * _D_BIG),
        (isz, wsz, small_v, _D_SMALL, 2 * _D_BIG + _D_SMALL),
        (igr, wgr, small_v, _D_SMALL, 2 * _D_BIG + 2 * _D_SMALL),
    ]
    for idx_hbm, w_hbm, rows_v, dcol, coff in tables:
      pltpu.sync_copy(idx_hbm.at[wid], idx_v)

      def body(c, carry, w_hbm=w_hbm, rows_v=rows_v, dcol=dcol, coff=coff):
        pltpu.async_copy(w_hbm.at[idx_v.at[c]], rows_v, sem).wait()
        pltpu.sync_copy(
            rows_v,
            out.at[pl.ds(base + c * _CHUNK, _CHUNK), pl.ds(coff, dcol)])
        return carry

      lax.fori_loop(0, _NCHUNK, body, None)

  return emb


_EMB = _build()


def kernel(product, customer, color, size, group,
           W_product, W_customer, W_color, W_size, W_group):
  def prep(i):
    return jnp.asarray(i, jnp.int32).reshape(_NW, _NCHUNK, _CHUNK)

  out = _EMB(prep(product), prep(customer), prep(color), prep(size),
             prep(group), W_product, W_customer, W_color, W_size, W_group)
  return out.reshape(1024, 20, _D_OUT)
